# Initial kernel scaffold; baseline (speedup 1.0000x reference)
import jax
import jax.numpy as jnp
from jax.experimental import pallas as pl


def kernel(x_n, x_e, edge_indices, all_embeddings, lin_n_w, lin_n_b, lin_e_w, lin_e_b, a, bias_n, bias_e):
    raise NotImplementedError("write your pallas kernel here")


# speedup vs baseline: 6.9641x; 6.9641x over previous
import jax
import jax.numpy as jnp
from jax import lax
from jax.experimental import pallas as pl
from jax.experimental.pallas import tpu as pltpu

B, N, K, W, D = 32, 256, 32, 100, 128
ALPHA = 0.2
E = N * K


def _lrelu(v):
    return jnp.where(v > 0, v, ALPHA * v)


def _attn_body(x_ref, idx_ref, wnT_ref, bn_ref, asel_ref, asum_ref, bias_ref,
               out_ref, p_ref):
    b = pl.program_id(0)

    @pl.when(b == 0)
    def _():
        cols = lax.broadcasted_iota(jnp.int32, (E, N), 1)
        p_ref[...] = (idx_ref[...] == cols).astype(jnp.float32)

    wx = jnp.dot(x_ref[0], wnT_ref[...],
                 preferred_element_type=jnp.float32) + bn_ref[...]

    g = jnp.dot(p_ref[...], wx, preferred_element_type=jnp.float32)
    g3 = g.reshape(N, K, D)

    e_sel = jnp.sum(g3 * asel_ref[...][None, :, :], axis=2)
    rj = lax.broadcasted_iota(jnp.int32, (K, K // 2), 0)
    rm = lax.broadcasted_iota(jnp.int32, (K, K // 2), 1)
    rmat = (rj // 2 == rm).astype(jnp.float32)
    e_hi = jnp.dot(e_sel, rmat, preferred_element_type=jnp.float32)
    s12 = jnp.sum(wx * asum_ref[...], axis=1, keepdims=True)
    e_lo = jnp.broadcast_to(s12, (N, K // 2))
    logits = _lrelu(jnp.concatenate([e_lo, e_hi], axis=1)) + bias_ref[...]

    mx = jnp.max(logits, axis=1, keepdims=True)
    p = jnp.exp(logits - mx)
    attn = p / jnp.sum(p, axis=1, keepdims=True)

    out_ref[0] = attn[:, :, None] * _lrelu(g3)


def kernel(x_n, x_e, edge_indices, all_embeddings, lin_n_w, lin_n_b,
           lin_e_w, lin_e_b, a, bias_n, bias_e):
    idx = edge_indices[0].reshape(E, 1)
    a2d = a.reshape(2, D)
    a_sel = a2d[jnp.arange(K) % 2]
    a_sum = (a2d[0] + a2d[1]).reshape(1, D)
    wnT = lin_n_w.T
    bn = lin_n_b.reshape(1, D)

    out = pl.pallas_call(
        _attn_body,
        grid=(B,),
        in_specs=[
            pl.BlockSpec((1, N, W), lambda b: (b, 0, 0)),
            pl.BlockSpec((E, 1), lambda b: (0, 0)),
            pl.BlockSpec((W, D), lambda b: (0, 0)),
            pl.BlockSpec((1, D), lambda b: (0, 0)),
            pl.BlockSpec((K, D), lambda b: (0, 0)),
            pl.BlockSpec((1, D), lambda b: (0, 0)),
            pl.BlockSpec((N, K), lambda b: (0, 0)),
        ],
        out_specs=pl.BlockSpec((1, N, K, D), lambda b: (b, 0, 0, 0)),
        out_shape=jax.ShapeDtypeStruct((B, N, K, D), jnp.float32),
        scratch_shapes=[pltpu.VMEM((E, N), jnp.float32)],
        compiler_params=pltpu.CompilerParams(
            dimension_semantics=("arbitrary",)),
    )(x_n, idx, wnT, bn, a_sel, a_sum, bias_n)
    return out
